# baseline (speedup 1.0000x reference)
import functools
import math

import jax
import jax.numpy as jnp
from jax import lax
from jax.experimental import pallas as pl
from jax.experimental.pallas import tpu as pltpu
from jax.experimental.pallas import tpu_sc as plsc

N_NODES = 10000
N_EDGES = 160000
F = 128
CUTOFF = 5.0
_LN2 = math.log(2.0)
_INV_LN2 = 1.0 / _LN2
_GAMMA = (127.0 / CUTOFF) ** 2
_STEP = CUTOFF / 127.0

EBLK = 2000
NBLK = 2000

NC = 2
NS = 16
NW = NC * NS
K = 64
ROWS = N_EDGES // K
ROWS_PAD = 2560
RPW = ROWS_PAD // NW
N_PAD = 10112
DUMP = N_PAD // NS


def _ssp(v):
    return jnp.maximum(v, 0.0) + jnp.log(1.0 + jnp.exp(-jnp.abs(v))) - _LN2


def _edge_kernel(d_ref, ef_ref, w1_ref, b1_ref, w2_ref, b2_ref, lp_ref, sp_ref):
    d = d_ref[...]
    cut = 0.5 * (jnp.cos((math.pi / CUTOFF) * d) + 1.0)
    c = lax.broadcasted_iota(jnp.int32, (EBLK, F), 1).astype(jnp.float32) * _STEP
    bf = jnp.exp(-_GAMMA * (d - c) ** 2)
    h1 = _ssp(jnp.dot(bf, w1_ref[...], preferred_element_type=jnp.float32) + b1_ref[...])
    h2 = _ssp(jnp.dot(h1, w2_ref[...], preferred_element_type=jnp.float32) + b2_ref[...])
    p = ef_ref[...] * h2 * cut
    lp_ref[...] = jnp.log(jnp.abs(p)) * _INV_LN2
    sp_ref[...] = jnp.where(p < 0.0, 1.0, 0.0)


def _node_kernel(x_ref, lx_ref, sx_ref):
    xv = x_ref[...]
    lx_ref[...] = jnp.log(jnp.abs(xv)) * _INV_LN2
    sx_ref[...] = jnp.where(xv < 0.0, 1.0, 0.0)


def _final_kernel(hl_ref, hn_ref, w3_ref, b3_ref, out_ref):
    hl = hl_ref[0] + hl_ref[1]
    n = hn_ref[0] + hn_ref[1]
    parity = n - 2.0 * jnp.floor(n * 0.5)
    sign = 1.0 - 2.0 * parity
    h = sign * jnp.exp(hl * _LN2)
    out_ref[...] = _ssp(jnp.dot(h, w3_ref[...], preferred_element_type=jnp.float32) + b3_ref[...])


def _sc_body(src_hbm, dst_hbm, lx_hbm, sx_hbm, lp_hbm, sp_hbm, zeros_hbm,
             hlog_hbm, hneg_hbm, sidx, didx, gbuf0, gbuf1, pbuf, acc,
             gsem0, gsem1):
    c = lax.axis_index("c")
    s = lax.axis_index("s")
    wid = s * NC + c

    nrows = jnp.where(wid == NW - 1, ROWS - RPW * (NW - 1), RPW)

    pltpu.sync_copy(src_hbm.at[pl.ds(RPW * wid, RPW)], sidx)
    pltpu.sync_copy(dst_hbm.at[pl.ds(RPW * wid, RPW)], didx)

    def phase(table_hbm, edge_hbm, out_hbm):
        pltpu.sync_copy(zeros_hbm, acc.at[pl.ds(DUMP * s, DUMP)])
        plsc.subcore_barrier()

        def halfstep(j, gb, gs):
            pltpu.sync_copy(edge_hbm.at[pl.ds((RPW * wid + j) * K, K)], pbuf)
            pltpu.make_async_copy(table_hbm.at[sidx.at[j]], gb, gs).wait()
            pltpu.sync_copy(gb, acc.at[didx.at[j]], add=True)
            pltpu.sync_copy(pbuf, acc.at[didx.at[j]], add=True)

        def body(jj, _):
            j0 = 2 * jj
            pltpu.async_copy(table_hbm.at[sidx.at[j0]], gbuf0, gsem0)
            pltpu.async_copy(table_hbm.at[sidx.at[j0 + 1]], gbuf1, gsem1)
            halfstep(j0, gbuf0, gsem0)
            halfstep(j0 + 1, gbuf1, gsem1)
            return 0

        lax.fori_loop(0, nrows // 2, body, 0)
        plsc.subcore_barrier()
        pltpu.sync_copy(acc.at[pl.ds(DUMP * s, DUMP)],
                        out_hbm.at[c, pl.ds(DUMP * s, DUMP)])
        plsc.subcore_barrier()

    phase(lx_hbm, lp_hbm, hlog_hbm)
    phase(sx_hbm, sp_hbm, hneg_hbm)


def kernel(x, edge_feat, dist, W1, b1, W2, b2, W3, b3, edge_index):
    idx2d = edge_index.astype(jnp.int32).reshape(2, ROWS, K)
    idx2d = jnp.pad(idx2d, ((0, 0), (0, ROWS_PAD - ROWS), (0, 0)))
    src2d, dst2d = idx2d[0], idx2d[1]
    d2 = dist[:, None]

    lp, sp = pl.pallas_call(
        _edge_kernel,
        grid=(N_EDGES // EBLK,),
        in_specs=[
            pl.BlockSpec((EBLK, 1), lambda i: (i, 0)),
            pl.BlockSpec((EBLK, F), lambda i: (i, 0)),
            pl.BlockSpec((F, F), lambda i: (0, 0)),
            pl.BlockSpec((1, F), lambda i: (0, 0)),
            pl.BlockSpec((F, F), lambda i: (0, 0)),
            pl.BlockSpec((1, F), lambda i: (0, 0)),
        ],
        out_specs=[
            pl.BlockSpec((EBLK, F), lambda i: (i, 0)),
            pl.BlockSpec((EBLK, F), lambda i: (i, 0)),
        ],
        out_shape=[
            jax.ShapeDtypeStruct((N_EDGES, F), jnp.float32),
            jax.ShapeDtypeStruct((N_EDGES, F), jnp.float32),
        ],
    )(d2, edge_feat, W1, b1[None, :], W2, b2[None, :])

    lx, sx = pl.pallas_call(
        _node_kernel,
        grid=(N_NODES // NBLK,),
        in_specs=[pl.BlockSpec((NBLK, F), lambda i: (i, 0))],
        out_specs=[
            pl.BlockSpec((NBLK, F), lambda i: (i, 0)),
            pl.BlockSpec((NBLK, F), lambda i: (i, 0)),
        ],
        out_shape=[
            jax.ShapeDtypeStruct((N_NODES, F), jnp.float32),
            jax.ShapeDtypeStruct((N_NODES, F), jnp.float32),
        ],
    )(x)

    zeros = jnp.zeros((DUMP, F), jnp.float32)
    mesh = plsc.VectorSubcoreMesh(core_axis_name="c", subcore_axis_name="s")
    sc = pl.kernel(
        _sc_body,
        out_type=[
            jax.ShapeDtypeStruct((NC, N_PAD, F), jnp.float32),
            jax.ShapeDtypeStruct((NC, N_PAD, F), jnp.float32),
        ],
        mesh=mesh,
        scratch_types=[
            pltpu.VMEM((RPW, K), jnp.int32),
            pltpu.VMEM((RPW, K), jnp.int32),
            pltpu.VMEM((K, F), jnp.float32),
            pltpu.VMEM((K, F), jnp.float32),
            pltpu.VMEM((K, F), jnp.float32),
            pltpu.VMEM_SHARED((N_PAD, F), jnp.float32),
            pltpu.SemaphoreType.DMA,
            pltpu.SemaphoreType.DMA,
        ],
    )
    hlog, hneg = sc(src2d, dst2d, lx, sx, lp, sp, zeros)

    out = pl.pallas_call(
        _final_kernel,
        grid=(N_NODES // NBLK,),
        in_specs=[
            pl.BlockSpec((NC, NBLK, F), lambda i: (0, i, 0)),
            pl.BlockSpec((NC, NBLK, F), lambda i: (0, i, 0)),
            pl.BlockSpec((F, F), lambda i: (0, 0)),
            pl.BlockSpec((1, F), lambda i: (0, 0)),
        ],
        out_specs=pl.BlockSpec((NBLK, F), lambda i: (i, 0)),
        out_shape=jax.ShapeDtypeStruct((N_NODES, F), jnp.float32),
    )(hlog, hneg, W3, b3[None, :])
    return out


# speedup vs baseline: 1.0924x; 1.0924x over previous
import functools
import math

import jax
import jax.numpy as jnp
from jax import lax
from jax.experimental import pallas as pl
from jax.experimental.pallas import tpu as pltpu
from jax.experimental.pallas import tpu_sc as plsc

N_NODES = 10000
N_EDGES = 160000
F = 128
CUTOFF = 5.0
_LN2 = math.log(2.0)
_INV_LN2 = 1.0 / _LN2
_GAMMA = (127.0 / CUTOFF) ** 2
_STEP = CUTOFF / 127.0

EBLK = 2000
NBLK = 2000

NC = 2
NS = 16
NW = NC * NS
K = 64
ROWS = N_EDGES // K
ROWS_PAD = 2560
RPW = ROWS_PAD // NW
N_PAD = 10112
DUMP = N_PAD // NS


_LOG2E = 1.0 / math.log(2.0)


def _ssp(v):
    e = jnp.exp2(jnp.abs(v) * -_LOG2E)
    return jnp.maximum(v, 0.0) + jnp.log2(1.0 + e) * _LN2 - _LN2


def _edge_kernel(d_ref, ef_ref, w1_ref, b1_ref, w2_ref, b2_ref, lp_ref, sp_ref):
    d = d_ref[...]
    cut = 0.5 * (jnp.cos((math.pi / CUTOFF) * d) + 1.0)
    c_row = lax.broadcasted_iota(jnp.int32, (1, F), 1).astype(jnp.float32) * _STEP
    a_top = (2.0 * _GAMMA * _LOG2E) * c_row
    a_bot = jnp.full((1, F), -_GAMMA * _LOG2E, jnp.float32)
    amat = jnp.concatenate([a_top, a_bot], axis=0)
    d_aug = jnp.concatenate([d, d * d], axis=1)
    arg = jnp.dot(d_aug, amat, preferred_element_type=jnp.float32) - (_GAMMA * _LOG2E) * c_row * c_row
    bf = jnp.exp2(arg)
    h1 = _ssp(jnp.dot(bf, w1_ref[...], preferred_element_type=jnp.float32) + b1_ref[...])
    h2 = _ssp(jnp.dot(h1, w2_ref[...], preferred_element_type=jnp.float32) + b2_ref[...])
    cutb = jnp.dot(cut, jnp.ones((1, F), jnp.float32), preferred_element_type=jnp.float32)
    p = ef_ref[...] * h2 * cutb
    lp_ref[...] = jnp.log2(jnp.abs(p))
    sp_ref[...] = jnp.where(p < 0.0, 1.0, 0.0)


def _node_kernel(x_ref, lx_ref, sx_ref):
    xv = x_ref[...]
    lx_ref[...] = jnp.log2(jnp.abs(xv))
    sx_ref[...] = jnp.where(xv < 0.0, 1.0, 0.0)


def _final_kernel(hl_ref, hn_ref, w3_ref, b3_ref, out_ref):
    hl = hl_ref[0] + hl_ref[1]
    n = hn_ref[0] + hn_ref[1]
    parity = n - 2.0 * jnp.floor(n * 0.5)
    sign = 1.0 - 2.0 * parity
    h = sign * jnp.exp2(hl)
    out_ref[...] = _ssp(jnp.dot(h, w3_ref[...], preferred_element_type=jnp.float32) + b3_ref[...])


HB = RPW // 2


def _sc_body(src_hbm, dst_hbm, lx_hbm, sx_hbm, lp_hbm, sp_hbm, zeros_hbm,
             hlog_hbm, hneg_hbm, sidx, didx, gbuf0, gbuf1, pbuf0, pbuf1, acc,
             gsem0, gsem1, psem0, psem1, ssem0, ssem1):
    c = lax.axis_index("c")
    s = lax.axis_index("s")
    wid = s * NC + c

    nrows = jnp.where(wid == NW - 1, ROWS - RPW * (NW - 1), RPW)

    def phase(table_hbm, edge_hbm, out_hbm):
        pltpu.sync_copy(zeros_hbm, acc.at[pl.ds(DUMP * s, DUMP)])
        plsc.subcore_barrier()

        def drain(gb, pb, ss):
            pltpu.make_async_copy(gb, acc.at[didx.at[0]], ss).wait()
            pltpu.make_async_copy(pb, acc.at[didx.at[0]], ss).wait()

        for h in range(2):
            n_h = jnp.clip(nrows - HB * h, 0, HB)

            @pl.when(n_h > 0)
            def _():
                pltpu.sync_copy(src_hbm.at[pl.ds(RPW * wid + HB * h, HB)], sidx)
                pltpu.sync_copy(dst_hbm.at[pl.ds(RPW * wid + HB * h, HB)], didx)

                def body(jj, _):
                    j0 = 2 * jj
                    r0 = RPW * wid + HB * h + j0

                    @pl.when(jj >= 1)
                    def _():
                        drain(gbuf0, pbuf0, ssem0)

                    pltpu.async_copy(table_hbm.at[sidx.at[j0]], gbuf0, gsem0)
                    pltpu.async_copy(edge_hbm.at[pl.ds(r0 * K, K)], pbuf0, psem0)

                    @pl.when(jj >= 1)
                    def _():
                        drain(gbuf1, pbuf1, ssem1)

                    pltpu.async_copy(table_hbm.at[sidx.at[j0 + 1]], gbuf1, gsem1)
                    pltpu.async_copy(edge_hbm.at[pl.ds((r0 + 1) * K, K)], pbuf1, psem1)

                    pltpu.make_async_copy(table_hbm.at[sidx.at[j0]], gbuf0, gsem0).wait()
                    pltpu.make_async_copy(edge_hbm.at[pl.ds(r0 * K, K)], pbuf0, psem0).wait()
                    pltpu.async_copy(gbuf0, acc.at[didx.at[j0]], ssem0, add=True)
                    pltpu.async_copy(pbuf0, acc.at[didx.at[j0]], ssem0, add=True)

                    pltpu.make_async_copy(table_hbm.at[sidx.at[j0 + 1]], gbuf1, gsem1).wait()
                    pltpu.make_async_copy(edge_hbm.at[pl.ds(r0 * K, K)], pbuf1, psem1).wait()
                    pltpu.async_copy(gbuf1, acc.at[didx.at[j0 + 1]], ssem1, add=True)
                    pltpu.async_copy(pbuf1, acc.at[didx.at[j0 + 1]], ssem1, add=True)
                    return 0

                lax.fori_loop(0, n_h // 2, body, 0)
                drain(gbuf0, pbuf0, ssem0)
                drain(gbuf1, pbuf1, ssem1)

        plsc.subcore_barrier()
        pltpu.sync_copy(acc.at[pl.ds(DUMP * s, DUMP)],
                        out_hbm.at[c, pl.ds(DUMP * s, DUMP)])
        plsc.subcore_barrier()

    phase(lx_hbm, lp_hbm, hlog_hbm)
    phase(sx_hbm, sp_hbm, hneg_hbm)


def kernel(x, edge_feat, dist, W1, b1, W2, b2, W3, b3, edge_index):
    idx2d = edge_index.astype(jnp.int32).reshape(2, ROWS, K)
    idx2d = jnp.pad(idx2d, ((0, 0), (0, ROWS_PAD - ROWS), (0, 0)))
    src2d, dst2d = idx2d[0], idx2d[1]
    d2 = dist[:, None]

    lp, sp = pl.pallas_call(
        _edge_kernel,
        grid=(N_EDGES // EBLK,),
        in_specs=[
            pl.BlockSpec((EBLK, 1), lambda i: (i, 0)),
            pl.BlockSpec((EBLK, F), lambda i: (i, 0)),
            pl.BlockSpec((F, F), lambda i: (0, 0)),
            pl.BlockSpec((1, F), lambda i: (0, 0)),
            pl.BlockSpec((F, F), lambda i: (0, 0)),
            pl.BlockSpec((1, F), lambda i: (0, 0)),
        ],
        out_specs=[
            pl.BlockSpec((EBLK, F), lambda i: (i, 0)),
            pl.BlockSpec((EBLK, F), lambda i: (i, 0)),
        ],
        out_shape=[
            jax.ShapeDtypeStruct((N_EDGES, F), jnp.float32),
            jax.ShapeDtypeStruct((N_EDGES, F), jnp.float32),
        ],
    )(d2, edge_feat, W1, b1[None, :], W2, b2[None, :])

    lx, sx = pl.pallas_call(
        _node_kernel,
        grid=(N_NODES // NBLK,),
        in_specs=[pl.BlockSpec((NBLK, F), lambda i: (i, 0))],
        out_specs=[
            pl.BlockSpec((NBLK, F), lambda i: (i, 0)),
            pl.BlockSpec((NBLK, F), lambda i: (i, 0)),
        ],
        out_shape=[
            jax.ShapeDtypeStruct((N_NODES, F), jnp.float32),
            jax.ShapeDtypeStruct((N_NODES, F), jnp.float32),
        ],
    )(x)

    zeros = jnp.zeros((DUMP, F), jnp.float32)
    mesh = plsc.VectorSubcoreMesh(core_axis_name="c", subcore_axis_name="s")
    sc = pl.kernel(
        _sc_body,
        out_type=[
            jax.ShapeDtypeStruct((NC, N_PAD, F), jnp.float32),
            jax.ShapeDtypeStruct((NC, N_PAD, F), jnp.float32),
        ],
        mesh=mesh,
        scratch_types=[
            pltpu.VMEM((HB, K), jnp.int32),
            pltpu.VMEM((HB, K), jnp.int32),
            pltpu.VMEM((K, F), jnp.float32),
            pltpu.VMEM((K, F), jnp.float32),
            pltpu.VMEM((K, F), jnp.float32),
            pltpu.VMEM((K, F), jnp.float32),
            pltpu.VMEM_SHARED((N_PAD, F), jnp.float32),
            pltpu.SemaphoreType.DMA,
            pltpu.SemaphoreType.DMA,
            pltpu.SemaphoreType.DMA,
            pltpu.SemaphoreType.DMA,
            pltpu.SemaphoreType.DMA,
            pltpu.SemaphoreType.DMA,
        ],
    )
    hlog, hneg = sc(src2d, dst2d, lx, sx, lp, sp, zeros)

    out = pl.pallas_call(
        _final_kernel,
        grid=(N_NODES // NBLK,),
        in_specs=[
            pl.BlockSpec((NC, NBLK, F), lambda i: (0, i, 0)),
            pl.BlockSpec((NC, NBLK, F), lambda i: (0, i, 0)),
            pl.BlockSpec((F, F), lambda i: (0, 0)),
            pl.BlockSpec((1, F), lambda i: (0, 0)),
        ],
        out_specs=pl.BlockSpec((NBLK, F), lambda i: (i, 0)),
        out_shape=jax.ShapeDtypeStruct((N_NODES, F), jnp.float32),
    )(hlog, hneg, W3, b3[None, :])
    return out


# speedup vs baseline: 1.6661x; 1.5253x over previous
import functools
import math

import jax
import jax.numpy as jnp
from jax import lax
from jax.experimental import pallas as pl
from jax.experimental.pallas import tpu as pltpu
from jax.experimental.pallas import tpu_sc as plsc

N_NODES = 10000
N_EDGES = 160000
F = 128
CUTOFF = 5.0
_LN2 = math.log(2.0)
_INV_LN2 = 1.0 / _LN2
_GAMMA = (127.0 / CUTOFF) ** 2
_STEP = CUTOFF / 127.0

EBLK = 3200
DBLK = EBLK // F
NBLK = 2000

NC = 2
NS = 16
NW = NC * NS
K = 64
ROWS = N_EDGES // K
HPAD = 1280
RPW = HPAD // NW
N_PAD = 10112
DUMP = N_PAD // NS


_LOG2E = 1.0 / math.log(2.0)


def _ssp(v):
    e = jnp.exp2(jnp.abs(v) * -_LOG2E)
    return jnp.maximum(v, 0.0) + jnp.log2(1.0 + e) * _LN2 - _LN2


def _lane_to_rows(mat):
    ones_row = jnp.ones((1, F), jnp.float32)
    pieces = [
        lax.dot_general(mat[r:r + 1, :], ones_row, (((0,), (0,)), ((), ())),
                        preferred_element_type=jnp.float32)
        for r in range(DBLK)
    ]
    return jnp.concatenate(pieces, axis=0)


def _edge_kernel(d_ref, ef_ref, w1_ref, b1_ref, w2_ref, b2_ref, lp_ref, sp_ref):
    dd = d_ref[0]
    cutd = 0.5 * (jnp.cos((math.pi / CUTOFF) * dd) + 1.0)
    db = _lane_to_rows(dd)
    cutb = _lane_to_rows(cutd)
    c_row = lax.broadcasted_iota(jnp.int32, (1, F), 1).astype(jnp.float32) * _STEP
    diff = db - c_row
    bf = jnp.exp2((-_GAMMA * _LOG2E) * diff * diff)
    h1 = _ssp(jnp.dot(bf, w1_ref[...], preferred_element_type=jnp.float32) + b1_ref[...])
    h2 = _ssp(jnp.dot(h1, w2_ref[...], preferred_element_type=jnp.float32) + b2_ref[...])
    p = ef_ref[...] * h2 * cutb
    lp_ref[...] = jnp.log2(jnp.abs(p))
    sp_ref[...] = jnp.where(p < 0.0, 1.0, 0.0)


def _node_kernel(x_ref, lx_ref, sx_ref):
    xv = x_ref[...]
    lx_ref[...] = jnp.log2(jnp.abs(xv))
    sx_ref[...] = jnp.where(xv < 0.0, 1.0, 0.0)


def _final_kernel(hla_ref, hna_ref, hlb_ref, hnb_ref, w3_ref, b3_ref, out_ref):
    hl = hla_ref[0] + hla_ref[1] + hlb_ref[0] + hlb_ref[1]
    n = hna_ref[0] + hna_ref[1] + hnb_ref[0] + hnb_ref[1]
    parity = n - 2.0 * jnp.floor(n * 0.5)
    sign = 1.0 - 2.0 * parity
    h = sign * jnp.exp2(hl)
    out_ref[...] = _ssp(jnp.dot(h, w3_ref[...], preferred_element_type=jnp.float32) + b3_ref[...])


HROWS = ROWS // 2


def _sc_body(src_hbm, dst_hbm, lx_hbm, sx_hbm, lp_hbm, sp_hbm,
             zeros_hbm, hlog_hbm, hneg_hbm, sidx, didx, gbuf0, gbuf1, pbuf0,
             pbuf1, acc, gsem0, gsem1, psem0, psem1, ssem0, ssem1):
    c = lax.axis_index("c")
    s = lax.axis_index("s")
    wid = s * NC + c

    nrows = jnp.where(wid == NW - 1, HROWS - RPW * (NW - 1), RPW)

    def phase(table_hbm, edge_hbm, out_hbm):
        pltpu.sync_copy(zeros_hbm, acc.at[pl.ds(DUMP * s, DUMP)])
        plsc.subcore_barrier()

        def drain(gb, pb, ss):
            pltpu.make_async_copy(gb, acc.at[didx.at[0]], ss).wait()
            pltpu.make_async_copy(pb, acc.at[didx.at[0]], ss).wait()

        pltpu.sync_copy(src_hbm.at[pl.ds(RPW * wid, RPW)], sidx)
        pltpu.sync_copy(dst_hbm.at[pl.ds(RPW * wid, RPW)], didx)

        def body(jj, _):
            j0 = 2 * jj
            r0 = RPW * wid + j0

            @pl.when(jj >= 1)
            def _():
                drain(gbuf0, pbuf0, ssem0)

            pltpu.async_copy(table_hbm.at[sidx.at[j0]], gbuf0, gsem0)
            pltpu.async_copy(edge_hbm.at[pl.ds(r0 * K, K)], pbuf0, psem0)

            @pl.when(jj >= 1)
            def _():
                drain(gbuf1, pbuf1, ssem1)

            pltpu.async_copy(table_hbm.at[sidx.at[j0 + 1]], gbuf1, gsem1)
            pltpu.async_copy(edge_hbm.at[pl.ds((r0 + 1) * K, K)], pbuf1, psem1)

            pltpu.make_async_copy(table_hbm.at[sidx.at[j0]], gbuf0, gsem0).wait()
            pltpu.make_async_copy(edge_hbm.at[pl.ds(r0 * K, K)], pbuf0, psem0).wait()
            pltpu.async_copy(gbuf0, acc.at[didx.at[j0]], ssem0, add=True)
            pltpu.async_copy(pbuf0, acc.at[didx.at[j0]], ssem0, add=True)

            pltpu.make_async_copy(table_hbm.at[sidx.at[j0 + 1]], gbuf1, gsem1).wait()
            pltpu.make_async_copy(edge_hbm.at[pl.ds(r0 * K, K)], pbuf1, psem1).wait()
            pltpu.async_copy(gbuf1, acc.at[didx.at[j0 + 1]], ssem1, add=True)
            pltpu.async_copy(pbuf1, acc.at[didx.at[j0 + 1]], ssem1, add=True)
            return 0

        lax.fori_loop(0, nrows // 2, body, 0)
        drain(gbuf0, pbuf0, ssem0)
        drain(gbuf1, pbuf1, ssem1)

        plsc.subcore_barrier()
        pltpu.sync_copy(acc.at[pl.ds(DUMP * s, DUMP)],
                        out_hbm.at[c, pl.ds(DUMP * s, DUMP)])
        plsc.subcore_barrier()

    phase(lx_hbm, lp_hbm, hlog_hbm)
    phase(sx_hbm, sp_hbm, hneg_hbm)


def kernel(x, edge_feat, dist, W1, b1, W2, b2, W3, b3, edge_index):
    idx2d = edge_index.astype(jnp.int32).reshape(2, ROWS, K)
    idx_a = jnp.pad(idx2d[:, :HROWS], ((0, 0), (0, HPAD - HROWS), (0, 0)))
    idx_b = jnp.pad(idx2d[:, HROWS:], ((0, 0), (0, HPAD - HROWS), (0, 0)))
    d2 = dist.reshape(N_EDGES // EBLK, DBLK, F)

    HE = N_EDGES // 2

    def edge_call(d2_h, ef_h):
        return pl.pallas_call(
            _edge_kernel,
            grid=(HE // EBLK,),
            in_specs=[
                pl.BlockSpec((1, DBLK, F), lambda i: (i, 0, 0)),
                pl.BlockSpec((EBLK, F), lambda i: (i, 0)),
                pl.BlockSpec((F, F), lambda i: (0, 0)),
                pl.BlockSpec((1, F), lambda i: (0, 0)),
                pl.BlockSpec((F, F), lambda i: (0, 0)),
                pl.BlockSpec((1, F), lambda i: (0, 0)),
            ],
            out_specs=[
                pl.BlockSpec((EBLK, F), lambda i: (i, 0)),
                pl.BlockSpec((EBLK, F), lambda i: (i, 0)),
            ],
            out_shape=[
                jax.ShapeDtypeStruct((HE, F), jnp.float32),
                jax.ShapeDtypeStruct((HE, F), jnp.float32),
            ],
        )(d2_h, ef_h, W1, b1[None, :], W2, b2[None, :])

    lp_a, sp_a = edge_call(d2[:HE // EBLK], edge_feat[:HE])

    lx, sx = pl.pallas_call(
        _node_kernel,
        grid=(N_NODES // NBLK,),
        in_specs=[pl.BlockSpec((NBLK, F), lambda i: (i, 0))],
        out_specs=[
            pl.BlockSpec((NBLK, F), lambda i: (i, 0)),
            pl.BlockSpec((NBLK, F), lambda i: (i, 0)),
        ],
        out_shape=[
            jax.ShapeDtypeStruct((N_NODES, F), jnp.float32),
            jax.ShapeDtypeStruct((N_NODES, F), jnp.float32),
        ],
    )(x)

    lp_b, sp_b = edge_call(d2[HE // EBLK:], edge_feat[HE:])

    zeros = jnp.zeros((DUMP, F), jnp.float32)
    mesh = plsc.VectorSubcoreMesh(core_axis_name="c", subcore_axis_name="s")

    def sc_call(src_h, dst_h, lp_h, sp_h):
        return pl.kernel(
            _sc_body,
            out_type=[
                jax.ShapeDtypeStruct((NC, N_PAD, F), jnp.float32),
                jax.ShapeDtypeStruct((NC, N_PAD, F), jnp.float32),
            ],
            mesh=mesh,
            scratch_types=[
                pltpu.VMEM((RPW, K), jnp.int32),
                pltpu.VMEM((RPW, K), jnp.int32),
                pltpu.VMEM((K, F), jnp.float32),
                pltpu.VMEM((K, F), jnp.float32),
                pltpu.VMEM((K, F), jnp.float32),
                pltpu.VMEM((K, F), jnp.float32),
                pltpu.VMEM_SHARED((N_PAD, F), jnp.float32),
                pltpu.SemaphoreType.DMA,
                pltpu.SemaphoreType.DMA,
                pltpu.SemaphoreType.DMA,
                pltpu.SemaphoreType.DMA,
                pltpu.SemaphoreType.DMA,
                pltpu.SemaphoreType.DMA,
            ],
        )(src_h, dst_h, lx, sx, lp_h, sp_h, zeros)

    hlog_a, hneg_a = sc_call(idx_a[0], idx_a[1], lp_a, sp_a)
    hlog_b, hneg_b = sc_call(idx_b[0], idx_b[1], lp_b, sp_b)

    out = pl.pallas_call(
        _final_kernel,
        grid=(N_NODES // NBLK,),
        in_specs=[
            pl.BlockSpec((NC, NBLK, F), lambda i: (0, i, 0)),
            pl.BlockSpec((NC, NBLK, F), lambda i: (0, i, 0)),
            pl.BlockSpec((NC, NBLK, F), lambda i: (0, i, 0)),
            pl.BlockSpec((NC, NBLK, F), lambda i: (0, i, 0)),
            pl.BlockSpec((F, F), lambda i: (0, 0)),
            pl.BlockSpec((1, F), lambda i: (0, 0)),
        ],
        out_specs=pl.BlockSpec((NBLK, F), lambda i: (i, 0)),
        out_shape=jax.ShapeDtypeStruct((N_NODES, F), jnp.float32),
    )(hlog_a, hneg_a, hlog_b, hneg_b, W3, b3[None, :])
    return out


# speedup vs baseline: 1.7607x; 1.0568x over previous
import functools
import math

import jax
import jax.numpy as jnp
from jax import lax
from jax.experimental import pallas as pl
from jax.experimental.pallas import tpu as pltpu
from jax.experimental.pallas import tpu_sc as plsc

N_NODES = 10000
N_EDGES = 160000
F = 128
CUTOFF = 5.0
_LN2 = math.log(2.0)
_INV_LN2 = 1.0 / _LN2
_GAMMA = (127.0 / CUTOFF) ** 2
_STEP = CUTOFF / 127.0

EBLK = 3200
DBLK = EBLK // F
NBLK = 2000

NC = 2
NS = 16
NW = NC * NS
K = 64
ROWS = N_EDGES // K
ROWS_PAD = 2560
RPW = ROWS_PAD // NW
N_PAD = 10112
DUMP = N_PAD // NS


_LOG2E = 1.0 / math.log(2.0)


def _ssp(v):
    e = jnp.exp2(jnp.abs(v) * -_LOG2E)
    return jnp.maximum(v, 0.0) + jnp.log2(1.0 + e) * _LN2 - _LN2


def _lane_to_rows(mat):
    ones_row = jnp.ones((1, F), jnp.float32)
    pieces = [
        lax.dot_general(mat[r:r + 1, :], ones_row, (((0,), (0,)), ((), ())),
                        preferred_element_type=jnp.float32)
        for r in range(DBLK)
    ]
    return jnp.concatenate(pieces, axis=0)


def _edge_kernel(d_ref, ef_ref, w1_ref, b1_ref, w2_ref, b2_ref, lp_ref, sp_ref):
    dd = d_ref[0]
    cutd = 0.5 * (jnp.cos((math.pi / CUTOFF) * dd) + 1.0)
    db = _lane_to_rows(dd)
    cutb = _lane_to_rows(cutd)
    c_row = lax.broadcasted_iota(jnp.int32, (1, F), 1).astype(jnp.float32) * _STEP
    diff = db - c_row
    bf = jnp.exp2((-_GAMMA * _LOG2E) * diff * diff)
    h1 = _ssp(jnp.dot(bf, w1_ref[...], preferred_element_type=jnp.float32) + b1_ref[...])
    h2 = _ssp(jnp.dot(h1, w2_ref[...], preferred_element_type=jnp.float32) + b2_ref[...])
    p = ef_ref[...] * h2 * cutb
    lp_ref[...] = jnp.log2(jnp.abs(p))
    sp_ref[...] = jnp.where(p < 0.0, 1.0, 0.0)


def _node_kernel(x_ref, lx_ref, sx_ref):
    xv = x_ref[...]
    lx_ref[...] = jnp.log2(jnp.abs(xv))
    sx_ref[...] = jnp.where(xv < 0.0, 1.0, 0.0)


def _final_kernel(hl_ref, hn_ref, w3_ref, b3_ref, out_ref):
    hl = hl_ref[0] + hl_ref[1]
    n = hn_ref[0] + hn_ref[1]
    parity = n - 2.0 * jnp.floor(n * 0.5)
    sign = 1.0 - 2.0 * parity
    h = sign * jnp.exp2(hl)
    out_ref[...] = _ssp(jnp.dot(h, w3_ref[...], preferred_element_type=jnp.float32) + b3_ref[...])


HB = RPW // 2


def _sc_body(src_hbm, dst_hbm, lx_hbm, sx_hbm, lp_hbm, sp_hbm, zeros_hbm,
             hlog_hbm, hneg_hbm, sidx, didx, gbuf0, gbuf1, pbuf0, pbuf1, acc,
             gsem0, gsem1, psem0, psem1, ssem0, ssem1):
    c = lax.axis_index("c")
    s = lax.axis_index("s")
    wid = s * NC + c

    nrows = jnp.where(wid == NW - 1, ROWS - RPW * (NW - 1), RPW)

    def phase(table_hbm, edge_hbm, out_hbm):
        pltpu.sync_copy(zeros_hbm, acc.at[pl.ds(DUMP * s, DUMP)])
        plsc.subcore_barrier()

        def drain(gb, pb, ss):
            pltpu.make_async_copy(gb, acc.at[didx.at[0]], ss).wait()
            pltpu.make_async_copy(pb, acc.at[didx.at[0]], ss).wait()

        for h in range(2):
            n_h = jnp.clip(nrows - HB * h, 0, HB)

            @pl.when(n_h > 0)
            def _():
                pltpu.sync_copy(src_hbm.at[pl.ds(RPW * wid + HB * h, HB)], sidx)
                pltpu.sync_copy(dst_hbm.at[pl.ds(RPW * wid + HB * h, HB)], didx)

                def body(jj, _):
                    j0 = 2 * jj
                    r0 = RPW * wid + HB * h + j0

                    @pl.when(jj >= 1)
                    def _():
                        drain(gbuf0, pbuf0, ssem0)

                    pltpu.async_copy(table_hbm.at[sidx.at[j0]], gbuf0, gsem0)
                    pltpu.async_copy(edge_hbm.at[pl.ds(r0 * K, K)], pbuf0, psem0)

                    @pl.when(jj >= 1)
                    def _():
                        drain(gbuf1, pbuf1, ssem1)

                    pltpu.async_copy(table_hbm.at[sidx.at[j0 + 1]], gbuf1, gsem1)
                    pltpu.async_copy(edge_hbm.at[pl.ds((r0 + 1) * K, K)], pbuf1, psem1)

                    pltpu.make_async_copy(table_hbm.at[sidx.at[j0]], gbuf0, gsem0).wait()
                    pltpu.make_async_copy(edge_hbm.at[pl.ds(r0 * K, K)], pbuf0, psem0).wait()
                    pltpu.async_copy(gbuf0, acc.at[didx.at[j0]], ssem0, add=True)
                    pltpu.async_copy(pbuf0, acc.at[didx.at[j0]], ssem0, add=True)

                    pltpu.make_async_copy(table_hbm.at[sidx.at[j0 + 1]], gbuf1, gsem1).wait()
                    pltpu.make_async_copy(edge_hbm.at[pl.ds(r0 * K, K)], pbuf1, psem1).wait()
                    pltpu.async_copy(gbuf1, acc.at[didx.at[j0 + 1]], ssem1, add=True)
                    pltpu.async_copy(pbuf1, acc.at[didx.at[j0 + 1]], ssem1, add=True)
                    return 0

                lax.fori_loop(0, n_h // 2, body, 0)
                drain(gbuf0, pbuf0, ssem0)
                drain(gbuf1, pbuf1, ssem1)

        plsc.subcore_barrier()
        pltpu.sync_copy(acc.at[pl.ds(DUMP * s, DUMP)],
                        out_hbm.at[c, pl.ds(DUMP * s, DUMP)])
        plsc.subcore_barrier()

    phase(lx_hbm, lp_hbm, hlog_hbm)
    phase(sx_hbm, sp_hbm, hneg_hbm)


def kernel(x, edge_feat, dist, W1, b1, W2, b2, W3, b3, edge_index):
    idx2d = edge_index.astype(jnp.int32).reshape(2, ROWS, K)
    idx2d = jnp.pad(idx2d, ((0, 0), (0, ROWS_PAD - ROWS), (0, 0)))
    src2d, dst2d = idx2d[0], idx2d[1]
    d2 = dist.reshape(N_EDGES // EBLK, DBLK, F)

    lp, sp = pl.pallas_call(
        _edge_kernel,
        grid=(N_EDGES // EBLK,),
        in_specs=[
            pl.BlockSpec((1, DBLK, F), lambda i: (i, 0, 0)),
            pl.BlockSpec((EBLK, F), lambda i: (i, 0)),
            pl.BlockSpec((F, F), lambda i: (0, 0)),
            pl.BlockSpec((1, F), lambda i: (0, 0)),
            pl.BlockSpec((F, F), lambda i: (0, 0)),
            pl.BlockSpec((1, F), lambda i: (0, 0)),
        ],
        out_specs=[
            pl.BlockSpec((EBLK, F), lambda i: (i, 0)),
            pl.BlockSpec((EBLK, F), lambda i: (i, 0)),
        ],
        out_shape=[
            jax.ShapeDtypeStruct((N_EDGES, F), jnp.float32),
            jax.ShapeDtypeStruct((N_EDGES, F), jnp.float32),
        ],
    )(d2, edge_feat, W1, b1[None, :], W2, b2[None, :])

    lx, sx = pl.pallas_call(
        _node_kernel,
        grid=(N_NODES // NBLK,),
        in_specs=[pl.BlockSpec((NBLK, F), lambda i: (i, 0))],
        out_specs=[
            pl.BlockSpec((NBLK, F), lambda i: (i, 0)),
            pl.BlockSpec((NBLK, F), lambda i: (i, 0)),
        ],
        out_shape=[
            jax.ShapeDtypeStruct((N_NODES, F), jnp.float32),
            jax.ShapeDtypeStruct((N_NODES, F), jnp.float32),
        ],
    )(x)

    zeros = jnp.zeros((DUMP, F), jnp.float32)
    mesh = plsc.VectorSubcoreMesh(core_axis_name="c", subcore_axis_name="s")
    sc = pl.kernel(
        _sc_body,
        out_type=[
            jax.ShapeDtypeStruct((NC, N_PAD, F), jnp.float32),
            jax.ShapeDtypeStruct((NC, N_PAD, F), jnp.float32),
        ],
        mesh=mesh,
        scratch_types=[
            pltpu.VMEM((HB, K), jnp.int32),
            pltpu.VMEM((HB, K), jnp.int32),
            pltpu.VMEM((K, F), jnp.float32),
            pltpu.VMEM((K, F), jnp.float32),
            pltpu.VMEM((K, F), jnp.float32),
            pltpu.VMEM((K, F), jnp.float32),
            pltpu.VMEM_SHARED((N_PAD, F), jnp.float32),
            pltpu.SemaphoreType.DMA,
            pltpu.SemaphoreType.DMA,
            pltpu.SemaphoreType.DMA,
            pltpu.SemaphoreType.DMA,
            pltpu.SemaphoreType.DMA,
            pltpu.SemaphoreType.DMA,
        ],
    )
    hlog, hneg = sc(src2d, dst2d, lx, sx, lp, sp, zeros)

    out = pl.pallas_call(
        _final_kernel,
        grid=(N_NODES // NBLK,),
        in_specs=[
            pl.BlockSpec((NC, NBLK, F), lambda i: (0, i, 0)),
            pl.BlockSpec((NC, NBLK, F), lambda i: (0, i, 0)),
            pl.BlockSpec((F, F), lambda i: (0, 0)),
            pl.BlockSpec((1, F), lambda i: (0, 0)),
        ],
        out_specs=pl.BlockSpec((NBLK, F), lambda i: (i, 0)),
        out_shape=jax.ShapeDtypeStruct((N_NODES, F), jnp.float32),
    )(hlog, hneg, W3, b3[None, :])
    return out
